# Initial kernel scaffold; baseline (speedup 1.0000x reference)
import jax
import jax.numpy as jnp
from jax.experimental import pallas as pl


def kernel(feat, edge_index, etype, norm, v_b, a_rb, loop_weight):
    raise NotImplementedError("write your pallas kernel here")


# speedup vs baseline: 2.2577x; 2.2577x over previous
import functools

import jax
import jax.numpy as jnp
from jax import lax
from jax.experimental import pallas as pl
from jax.experimental.pallas import tpu as pltpu
from jax.experimental.pallas import tpu_sc as plsc

N = 10000
E = 320000
IN_DIM = 128
OUT_DIM = 128
NUM_RELS = 16
NUM_BASES = 4

NUM_WORKERS = 32
CHUNK = 128
CHUNKS_PER_WORKER = -(-E // (NUM_WORKERS * CHUNK))
EDGES_PER_WORKER = CHUNKS_PER_WORKER * CHUNK
E_PAD = EDGES_PER_WORKER * NUM_WORKERS
N_PAD = 10240
ROWS_PER_TILE = N_PAD // 16

N_BLK = 1000


def _proj_body(a_rb_ref, v_b_ref, feat_ref, out_ref):
    f = feat_ref[...]
    a = a_rb_ref[...]
    hb = [jnp.dot(f, v_b_ref[b], preferred_element_type=jnp.float32)
          for b in range(NUM_BASES)]
    for r in range(NUM_RELS):
        acc = a[r:r + 1, 0:1] * hb[0]
        for b in range(1, NUM_BASES):
            acc = acc + a[r:r + 1, b:b + 1] * hb[b]
        out_ref[r] = acc


def _final_body(lw_ref, feat_ref, parts_ref, out_ref):
    h = jnp.dot(feat_ref[...], lw_ref[...], preferred_element_type=jnp.float32)
    out_ref[...] = jnp.maximum(parts_ref[0] + parts_ref[1] + h, 0.0)


def _sc_body(gidx_hbm, dst_hbm, norm_hbm, hproj_hbm, zeros_hbm, out_hbm,
             gidx_v, dst_v, norm_v, rows_v, acc, sem):
    c = lax.axis_index("c")
    s = lax.axis_index("s")
    wid = c * 16 + s

    pltpu.sync_copy(zeros_hbm.at[pl.ds(s * ROWS_PER_TILE, ROWS_PER_TILE)],
                    acc.at[pl.ds(s * ROWS_PER_TILE, ROWS_PER_TILE)])
    plsc.subcore_barrier()

    base = wid * EDGES_PER_WORKER

    def chunk_body(i, carry):
        off = base + i * CHUNK
        pltpu.sync_copy(gidx_hbm.at[pl.ds(off, CHUNK)], gidx_v)
        pltpu.sync_copy(dst_hbm.at[pl.ds(off, CHUNK)], dst_v)
        pltpu.sync_copy(norm_hbm.at[pl.ds(off, CHUNK)], norm_v)
        pltpu.async_copy(hproj_hbm.at[gidx_v], rows_v, sem).wait()

        def scale_body(g, carry2):
            n16 = norm_v[pl.ds(g * 16, 16)]
            for l in range(16):
                nl = jnp.broadcast_to(n16[l], (16,))
                row = g * 16 + l
                for k in range(OUT_DIM // 16):
                    sl = pl.ds(k * 16, 16)
                    rows_v[row, sl] = rows_v[row, sl] * nl
            return carry2

        lax.fori_loop(0, CHUNK // 16, scale_body, 0)
        pltpu.sync_copy(rows_v, acc.at[dst_v], add=True)
        return carry

    lax.fori_loop(0, CHUNKS_PER_WORKER, chunk_body, 0)
    plsc.subcore_barrier()

    pltpu.sync_copy(acc.at[pl.ds(s * ROWS_PER_TILE, ROWS_PER_TILE)],
                    out_hbm.at[c, pl.ds(s * ROWS_PER_TILE, ROWS_PER_TILE)])


def kernel(feat, edge_index, etype, norm, v_b, a_rb, loop_weight):
    nb = N // N_BLK

    h_proj = pl.pallas_call(
        _proj_body,
        grid=(nb,),
        in_specs=[
            pl.BlockSpec((NUM_RELS, NUM_BASES), lambda i: (0, 0)),
            pl.BlockSpec((NUM_BASES, IN_DIM, OUT_DIM), lambda i: (0, 0, 0)),
            pl.BlockSpec((N_BLK, IN_DIM), lambda i: (i, 0)),
        ],
        out_specs=pl.BlockSpec((NUM_RELS, N_BLK, OUT_DIM), lambda i: (0, i, 0)),
        out_shape=jax.ShapeDtypeStruct((NUM_RELS, N, OUT_DIM), jnp.float32),
    )(a_rb, v_b, feat)
    h_proj = h_proj.reshape(NUM_RELS * N, OUT_DIM)

    src = edge_index[0]
    dst = edge_index[1]
    gidx = etype * N + src
    pad = E_PAD - E
    gidx_p = jnp.concatenate([gidx, jnp.zeros((pad,), jnp.int32)])
    dst_p = jnp.concatenate([dst, jnp.zeros((pad,), jnp.int32)])
    norm_p = jnp.concatenate([norm[:, 0], jnp.zeros((pad,), jnp.float32)])
    zeros = jnp.zeros((N_PAD, OUT_DIM), jnp.float32)

    mesh = plsc.VectorSubcoreMesh(core_axis_name="c", subcore_axis_name="s")
    sc_call = functools.partial(
        pl.kernel,
        mesh=mesh,
        out_type=jax.ShapeDtypeStruct((2, N_PAD, OUT_DIM), jnp.float32),
        scratch_types=[
            pltpu.VMEM((CHUNK,), jnp.int32),
            pltpu.VMEM((CHUNK,), jnp.int32),
            pltpu.VMEM((CHUNK,), jnp.float32),
            pltpu.VMEM((CHUNK, OUT_DIM), jnp.float32),
            pltpu.VMEM_SHARED((N_PAD, OUT_DIM), jnp.float32),
            pltpu.SemaphoreType.DMA,
        ],
    )(_sc_body)
    parts = sc_call(gidx_p, dst_p, norm_p, h_proj, zeros)

    out = pl.pallas_call(
        _final_body,
        grid=(nb,),
        in_specs=[
            pl.BlockSpec((IN_DIM, OUT_DIM), lambda i: (0, 0)),
            pl.BlockSpec((N_BLK, IN_DIM), lambda i: (i, 0)),
            pl.BlockSpec((2, N_BLK, OUT_DIM), lambda i: (0, i, 0)),
        ],
        out_specs=pl.BlockSpec((N_BLK, OUT_DIM), lambda i: (i, 0)),
        out_shape=jax.ShapeDtypeStruct((N, OUT_DIM), jnp.float32),
    )(loop_weight, feat, parts)
    return out


# speedup vs baseline: 2.5781x; 1.1420x over previous
import functools

import jax
import jax.numpy as jnp
from jax import lax
from jax.experimental import pallas as pl
from jax.experimental.pallas import tpu as pltpu
from jax.experimental.pallas import tpu_sc as plsc

N = 10000
E = 320000
IN_DIM = 128
OUT_DIM = 128
NUM_RELS = 16
NUM_BASES = 4

NUM_WORKERS = 32
CHUNK = 128
CHUNKS_PER_WORKER = 80
GROUP = 16
NGROUPS = CHUNKS_PER_WORKER // GROUP
EDGES_PER_WORKER = CHUNKS_PER_WORKER * CHUNK
E_PAD = EDGES_PER_WORKER * NUM_WORKERS
N_PAD = 10240
ROWS_PER_TILE = N_PAD // 16

N_BLK = 1000


def _proj_body(a_rb_ref, v_b_ref, feat_ref, out_ref):
    f = feat_ref[...]
    a = a_rb_ref[...]
    hb = [jnp.dot(f, v_b_ref[b], preferred_element_type=jnp.float32)
          for b in range(NUM_BASES)]
    for r in range(NUM_RELS):
        acc = a[r:r + 1, 0:1] * hb[0]
        for b in range(1, NUM_BASES):
            acc = acc + a[r:r + 1, b:b + 1] * hb[b]
        out_ref[r] = acc


def _final_body(lw_ref, feat_ref, parts_ref, out_ref):
    h = jnp.dot(feat_ref[...], lw_ref[...], preferred_element_type=jnp.float32)
    out_ref[...] = jnp.maximum(parts_ref[0] + parts_ref[1] + h, 0.0)


def _sc_body(gidx_hbm, dst_hbm, norm_hbm, hproj_hbm, zeros_hbm, out_hbm,
             gidx_v, dst_v, norm_v, rows0, rows1, acc, sem):
    c = lax.axis_index("c")
    s = lax.axis_index("s")
    wid = c * 16 + s

    pltpu.sync_copy(zeros_hbm.at[pl.ds(s * ROWS_PER_TILE, ROWS_PER_TILE)],
                    acc.at[pl.ds(s * ROWS_PER_TILE, ROWS_PER_TILE)])

    plsc.subcore_barrier()

    def scale_scatter(i, buf):
        def scale_body(g, carry2):
            n16 = norm_v[i, pl.ds(g * 16, 16)]
            for l in range(16):
                nl = jnp.broadcast_to(n16[l], (16,))
                row = g * 16 + l
                for k in range(OUT_DIM // 16):
                    sl = pl.ds(k * 16, 16)
                    buf[row, sl] = buf[row, sl] * nl
            return carry2

        lax.fori_loop(0, CHUNK // 16, scale_body, 0)
        pltpu.sync_copy(buf, acc.at[dst_v.at[i]], add=True)

    def group_body(gi, carry):
        base_chunk = gi * GROUP
        pltpu.sync_copy(gidx_hbm.at[wid, pl.ds(base_chunk, GROUP)], gidx_v)
        pltpu.sync_copy(dst_hbm.at[wid, pl.ds(base_chunk, GROUP)], dst_v)
        pltpu.sync_copy(norm_hbm.at[wid, pl.ds(base_chunk, GROUP)], norm_v)
        pltpu.async_copy(hproj_hbm.at[gidx_v.at[0]], rows0, sem)

        def pair_body(j, carry2):
            i0 = 2 * j
            pltpu.make_async_copy(hproj_hbm.at[gidx_v.at[i0]], rows0, sem).wait()
            pltpu.async_copy(hproj_hbm.at[gidx_v.at[i0 + 1]], rows1, sem)
            scale_scatter(i0, rows0)
            pltpu.make_async_copy(hproj_hbm.at[gidx_v.at[i0 + 1]], rows1,
                                  sem).wait()

            @pl.when(j < GROUP // 2 - 1)
            def _():
                pltpu.async_copy(hproj_hbm.at[gidx_v.at[i0 + 2]], rows0, sem)

            scale_scatter(i0 + 1, rows1)
            return carry2

        lax.fori_loop(0, GROUP // 2, pair_body, 0)
        return carry

    lax.fori_loop(0, NGROUPS, group_body, 0)
    plsc.subcore_barrier()

    pltpu.sync_copy(acc.at[pl.ds(s * ROWS_PER_TILE, ROWS_PER_TILE)],
                    out_hbm.at[c, pl.ds(s * ROWS_PER_TILE, ROWS_PER_TILE)])


def kernel(feat, edge_index, etype, norm, v_b, a_rb, loop_weight):
    nb = N // N_BLK

    h_proj = pl.pallas_call(
        _proj_body,
        grid=(nb,),
        in_specs=[
            pl.BlockSpec((NUM_RELS, NUM_BASES), lambda i: (0, 0)),
            pl.BlockSpec((NUM_BASES, IN_DIM, OUT_DIM), lambda i: (0, 0, 0)),
            pl.BlockSpec((N_BLK, IN_DIM), lambda i: (i, 0)),
        ],
        out_specs=pl.BlockSpec((NUM_RELS, N_BLK, OUT_DIM), lambda i: (0, i, 0)),
        out_shape=jax.ShapeDtypeStruct((NUM_RELS, N, OUT_DIM), jnp.float32),
    )(a_rb, v_b, feat)
    h_proj = h_proj.reshape(NUM_RELS * N, OUT_DIM)

    src = edge_index[0]
    dst = edge_index[1]
    gidx = etype * N + src
    pad = E_PAD - E
    shp = (NUM_WORKERS, CHUNKS_PER_WORKER, CHUNK)
    gidx_p = jnp.concatenate([gidx, jnp.zeros((pad,), jnp.int32)]).reshape(shp)
    dst_p = jnp.concatenate([dst, jnp.zeros((pad,), jnp.int32)]).reshape(shp)
    norm_p = jnp.concatenate([norm[:, 0], jnp.zeros((pad,), jnp.float32)]).reshape(shp)
    zeros = jnp.zeros((N_PAD, OUT_DIM), jnp.float32)

    mesh = plsc.VectorSubcoreMesh(core_axis_name="c", subcore_axis_name="s")
    sc_call = functools.partial(
        pl.kernel,
        mesh=mesh,
        out_type=jax.ShapeDtypeStruct((2, N_PAD, OUT_DIM), jnp.float32),
        scratch_types=[
            pltpu.VMEM((GROUP, CHUNK), jnp.int32),
            pltpu.VMEM((GROUP, CHUNK), jnp.int32),
            pltpu.VMEM((GROUP, CHUNK), jnp.float32),
            pltpu.VMEM((CHUNK, OUT_DIM), jnp.float32),
            pltpu.VMEM((CHUNK, OUT_DIM), jnp.float32),
            pltpu.VMEM_SHARED((N_PAD, OUT_DIM), jnp.float32),
            pltpu.SemaphoreType.DMA,
        ],
    )(_sc_body)
    parts = sc_call(gidx_p, dst_p, norm_p, h_proj, zeros)

    out = pl.pallas_call(
        _final_body,
        grid=(nb,),
        in_specs=[
            pl.BlockSpec((IN_DIM, OUT_DIM), lambda i: (0, 0)),
            pl.BlockSpec((N_BLK, IN_DIM), lambda i: (i, 0)),
            pl.BlockSpec((2, N_BLK, OUT_DIM), lambda i: (0, i, 0)),
        ],
        out_specs=pl.BlockSpec((N_BLK, OUT_DIM), lambda i: (i, 0)),
        out_shape=jax.ShapeDtypeStruct((N, OUT_DIM), jnp.float32),
    )(loop_weight, feat, parts)
    return out
